# Initial kernel scaffold; baseline (speedup 1.0000x reference)
import jax
import jax.numpy as jnp
from jax.experimental import pallas as pl


def kernel(x, edge_index, W_l1, W_r1, b1, W_l2, W_r2, b2):
    raise NotImplementedError("write your pallas kernel here")


# speedup vs baseline: 8.4188x; 8.4188x over previous
import functools

import jax
import jax.numpy as jnp
from jax import lax
from jax.experimental import pallas as pl
from jax.experimental.pallas import tpu as pltpu
from jax.experimental.pallas import tpu_sc as plsc

N = 10000
NPAD = 10240
E = 320000
D = 64
IN_D = 128
NW = 32
CH = 80
NCH = 125
RPT = 640

_f32 = jnp.float32


def _proj1_body(x_ref, w_ref, b_ref, p_ref, r_ref):
    xw = jnp.dot(x_ref[...], w_ref[...], preferred_element_type=_f32)
    p_ref[...] = xw[:, :D]
    r_ref[...] = xw[:, D:] + b_ref[...]


def _mid_body(s_ref, d_ref, r1_ref, w_ref, b_ref, p_ref, r_ref):
    s = s_ref[0, :N, :] + s_ref[1, :N, :]
    deg = d_ref[0, :N, :] + d_ref[1, :N, :]
    agg = s / jnp.maximum(deg, 1.0)
    h1 = jnp.maximum(agg + r1_ref[...], 0.0)
    hw = jnp.dot(h1, w_ref[...], preferred_element_type=_f32)
    p_ref[...] = hw[:, :D]
    r_ref[...] = hw[:, D:] + b_ref[...]


def _fin_body(s_ref, d_ref, r2_ref, o_ref):
    s = s_ref[0, :N, :] + s_ref[1, :N, :]
    deg = d_ref[0, :N, :] + d_ref[1, :N, :]
    agg = s / jnp.maximum(deg, 1.0)
    o_ref[...] = jnp.maximum(agg + r2_ref[...], 0.0)


_proj1 = pl.pallas_call(
    _proj1_body,
    out_shape=(jax.ShapeDtypeStruct((N, D), _f32),
               jax.ShapeDtypeStruct((N, D), _f32)),
)

_mid = pl.pallas_call(
    _mid_body,
    out_shape=(jax.ShapeDtypeStruct((N, D), _f32),
               jax.ShapeDtypeStruct((N, D), _f32)),
)

_fin = pl.pallas_call(
    _fin_body,
    out_shape=jax.ShapeDtypeStruct((N, D), _f32),
)


def _make_sc_pass(compute_deg):
    mesh = plsc.VectorSubcoreMesh(core_axis_name="c", subcore_axis_name="s")
    out_type = [jax.ShapeDtypeStruct((2, NPAD, D), _f32)]
    scratch = [
        pltpu.VMEM((NCH, CH), jnp.int32),
        pltpu.VMEM((NCH, CH), jnp.int32),
        pltpu.VMEM((CH, D), _f32),
        pltpu.VMEM((128, D), _f32),
        pltpu.VMEM_SHARED((NPAD, D), _f32),
        pltpu.SemaphoreType.DMA,
    ]
    if compute_deg:
        out_type.append(jax.ShapeDtypeStruct((2, NPAD), _f32))
        scratch += [
            pltpu.VMEM((CH,), _f32),
            pltpu.VMEM((RPT,), _f32),
            pltpu.VMEM_SHARED((NPAD,), _f32),
        ]

    def body(p_hbm, src_hbm, dst_hbm, s_out, *rest):
        if compute_deg:
            (deg_out, src_v, dst_v, rows_v, zbuf, acc_sh, sem,
             ones_v, dzbuf, deg_sh) = rest
        else:
            src_v, dst_v, rows_v, zbuf, acc_sh, sem = rest
        cid = lax.axis_index("c")
        sid = lax.axis_index("s")
        wid = sid * 2 + cid
        base = sid * RPT

        idx_cp1 = pltpu.async_copy(src_hbm.at[wid], src_v, sem)
        idx_cp2 = pltpu.async_copy(dst_hbm.at[wid], dst_v, sem)

        def zrow(r, carry):
            for c in range(D // 16):
                zbuf[r, pl.ds(c * 16, 16)] = jnp.zeros((16,), _f32)
            return carry
        lax.fori_loop(0, 128, zrow, 0)
        for k in range(RPT // 128):
            pltpu.sync_copy(zbuf, acc_sh.at[pl.ds(base + k * 128, 128)])
        if compute_deg:
            def zdeg(i, carry):
                dzbuf[pl.ds(i * 16, 16)] = jnp.zeros((16,), _f32)
                return carry
            lax.fori_loop(0, RPT // 16, zdeg, 0)
            for i in range(CH // 16):
                ones_v[pl.ds(i * 16, 16)] = jnp.ones((16,), _f32)
            pltpu.sync_copy(dzbuf, deg_sh.at[pl.ds(base, RPT)])
        idx_cp1.wait()
        idx_cp2.wait()
        plsc.subcore_barrier()

        def chunk(j, carry):
            pltpu.async_copy(p_hbm.at[src_v.at[j]], rows_v, sem).wait()
            pltpu.sync_copy(rows_v, acc_sh.at[dst_v.at[j]], add=True)
            if compute_deg:
                pltpu.sync_copy(ones_v, deg_sh.at[dst_v.at[j]], add=True)
            return carry
        lax.fori_loop(0, NCH, chunk, 0)

        plsc.subcore_barrier()
        pltpu.sync_copy(acc_sh.at[pl.ds(base, RPT)],
                        s_out.at[cid, pl.ds(base, RPT)])
        if compute_deg:
            pltpu.sync_copy(deg_sh.at[pl.ds(base, RPT)],
                            deg_out.at[cid, pl.ds(base, RPT)])

    return pl.kernel(body, mesh=mesh, out_type=tuple(out_type),
                     scratch_types=scratch,
                     compiler_params=pltpu.CompilerParams(
                         use_tc_tiling_on_sc=False))


_sc_pass_deg = _make_sc_pass(True)
_sc_pass = _make_sc_pass(False)


def kernel(x, edge_index, W_l1, W_r1, b1, W_l2, W_r2, b2):
    src = edge_index[0].astype(jnp.int32).reshape(NW, NCH, CH)
    dst = edge_index[1].astype(jnp.int32).reshape(NW, NCH, CH)
    wcat1 = jnp.concatenate([W_l1, W_r1], axis=1)
    wcat2 = jnp.concatenate([W_l2, W_r2], axis=1)

    p1, r1 = _proj1(x, wcat1, b1.reshape(1, D))
    s1, degp = _sc_pass_deg(p1, src, dst)
    degc = degp.reshape(2, NPAD, 1)
    p2, r2 = _mid(s1, degc, r1, wcat2, b2.reshape(1, D))
    s2, = _sc_pass(p2, src, dst)
    return _fin(s2, degc, r2)


# speedup vs baseline: 15.2801x; 1.8150x over previous
import functools

import jax
import jax.numpy as jnp
from jax import lax
from jax.experimental import pallas as pl
from jax.experimental.pallas import tpu as pltpu
from jax.experimental.pallas import tpu_sc as plsc

N = 10000
NPAD = 10240
E = 320000
D = 64
IN_D = 128
NW = 32
CH = 80
NCH = 125
G = 5
NGRP = NCH // G
RPT = 640

_f32 = jnp.float32


def _proj1_body(x_ref, w_ref, b_ref, p_ref, r_ref):
    xw = jnp.dot(x_ref[...], w_ref[...], preferred_element_type=_f32)
    p_ref[...] = xw[:, :D]
    r_ref[...] = xw[:, D:] + b_ref[...]


def _mid_body(s_ref, d_ref, r1_ref, w_ref, b_ref, p_ref, r_ref):
    s = s_ref[0, :N, :] + s_ref[1, :N, :]
    deg = d_ref[0, :N, :] + d_ref[1, :N, :]
    agg = s / jnp.maximum(deg, 1.0)
    h1 = jnp.maximum(agg + r1_ref[...], 0.0)
    hw = jnp.dot(h1, w_ref[...], preferred_element_type=_f32)
    p_ref[...] = hw[:, :D]
    r_ref[...] = hw[:, D:] + b_ref[...]


def _fin_body(s_ref, d_ref, r2_ref, o_ref):
    s = s_ref[0, :N, :] + s_ref[1, :N, :]
    deg = d_ref[0, :N, :] + d_ref[1, :N, :]
    agg = s / jnp.maximum(deg, 1.0)
    o_ref[...] = jnp.maximum(agg + r2_ref[...], 0.0)


_proj1 = pl.pallas_call(
    _proj1_body,
    out_shape=(jax.ShapeDtypeStruct((N, D), _f32),
               jax.ShapeDtypeStruct((N, D), _f32)),
)

_mid = pl.pallas_call(
    _mid_body,
    out_shape=(jax.ShapeDtypeStruct((N, D), _f32),
               jax.ShapeDtypeStruct((N, D), _f32)),
)

_fin = pl.pallas_call(
    _fin_body,
    out_shape=jax.ShapeDtypeStruct((N, D), _f32),
)


def _make_sc_pass(compute_deg):
    mesh = plsc.VectorSubcoreMesh(core_axis_name="c", subcore_axis_name="s")
    out_type = [jax.ShapeDtypeStruct((2, NPAD, D), _f32)]
    scratch = [
        pltpu.VMEM((NCH, CH), jnp.int32),
        pltpu.VMEM((NCH, CH), jnp.int32),
        pltpu.VMEM((2 * G, CH, D), _f32),
        pltpu.VMEM((128, D), _f32),
        pltpu.VMEM_SHARED((NPAD, D), _f32),
        pltpu.SemaphoreType.DMA,
        pltpu.SemaphoreType.DMA,
    ]
    if compute_deg:
        out_type.append(jax.ShapeDtypeStruct((2, NPAD), _f32))
        scratch += [
            pltpu.VMEM((CH,), _f32),
            pltpu.VMEM((RPT,), _f32),
            pltpu.VMEM_SHARED((NPAD,), _f32),
        ]

    def body(p_hbm, src_hbm, dst_hbm, s_out, *rest):
        if compute_deg:
            (deg_out, src_v, dst_v, rows_v, zbuf, acc_sh, sem_a, sem_b,
             ones_v, dzbuf, deg_sh) = rest
        else:
            src_v, dst_v, rows_v, zbuf, acc_sh, sem_a, sem_b = rest
        cid = lax.axis_index("c")
        sid = lax.axis_index("s")
        wid = sid * 2 + cid
        base = sid * RPT

        idx_cp1 = pltpu.async_copy(src_hbm.at[wid], src_v, sem_a)
        idx_cp2 = pltpu.async_copy(dst_hbm.at[wid], dst_v, sem_b)

        def zrow(r, carry):
            for c in range(D // 16):
                zbuf[r, pl.ds(c * 16, 16)] = jnp.zeros((16,), _f32)
            return carry
        lax.fori_loop(0, 128, zrow, 0)
        for k in range(RPT // 128):
            pltpu.sync_copy(zbuf, acc_sh.at[pl.ds(base + k * 128, 128)])
        if compute_deg:
            def zdeg(i, carry):
                dzbuf[pl.ds(i * 16, 16)] = jnp.zeros((16,), _f32)
                return carry
            lax.fori_loop(0, RPT // 16, zdeg, 0)
            for i in range(CH // 16):
                ones_v[pl.ds(i * 16, 16)] = jnp.ones((16,), _f32)
            pltpu.sync_copy(dzbuf, deg_sh.at[pl.ds(base, RPT)])
        idx_cp1.wait()
        idx_cp2.wait()
        plsc.subcore_barrier()

        def fire(g, boff, sem):
            for b in range(G):
                pltpu.async_copy(p_hbm.at[src_v.at[g * G + b]],
                                 rows_v.at[boff + b], sem)

        def drain_and_scatter(g, boff, sem):
            for b in range(G):
                pltpu.make_async_copy(p_hbm.at[src_v.at[g * G + b]],
                                      rows_v.at[boff + b], sem).wait()
            for b in range(G):
                c = g * G + b
                pltpu.sync_copy(rows_v.at[boff + b], acc_sh.at[dst_v.at[c]],
                                add=True)
                if compute_deg:
                    pltpu.sync_copy(ones_v, deg_sh.at[dst_v.at[c]], add=True)

        fire(0, 0, sem_a)

        def pair(t, carry):
            ga = 2 * t
            fire(ga + 1, G, sem_b)
            drain_and_scatter(ga, 0, sem_a)
            fire(ga + 2, 0, sem_a)
            drain_and_scatter(ga + 1, G, sem_b)
            return carry
        lax.fori_loop(0, (NGRP - 1) // 2, pair, 0)
        drain_and_scatter(NGRP - 1, 0, sem_a)

        plsc.subcore_barrier()
        pltpu.sync_copy(acc_sh.at[pl.ds(base, RPT)],
                        s_out.at[cid, pl.ds(base, RPT)])
        if compute_deg:
            pltpu.sync_copy(deg_sh.at[pl.ds(base, RPT)],
                            deg_out.at[cid, pl.ds(base, RPT)])

    return pl.kernel(body, mesh=mesh, out_type=tuple(out_type),
                     scratch_types=scratch,
                     compiler_params=pltpu.CompilerParams(
                         use_tc_tiling_on_sc=False))


_sc_pass_deg = _make_sc_pass(True)
_sc_pass = _make_sc_pass(False)


def kernel(x, edge_index, W_l1, W_r1, b1, W_l2, W_r2, b2):
    src = edge_index[0].astype(jnp.int32).reshape(NW, NCH, CH)
    dst = edge_index[1].astype(jnp.int32).reshape(NW, NCH, CH)
    wcat1 = jnp.concatenate([W_l1, W_r1], axis=1)
    wcat2 = jnp.concatenate([W_l2, W_r2], axis=1)

    p1, r1 = _proj1(x, wcat1, b1.reshape(1, D))
    s1, degp = _sc_pass_deg(p1, src, dst)
    degc = degp.reshape(2, NPAD, 1)
    p2, r2 = _mid(s1, degc, r1, wcat2, b2.reshape(1, D))
    s2, = _sc_pass(p2, src, dst)
    return _fin(s2, degc, r2)


# speedup vs baseline: 17.8156x; 1.1659x over previous
import functools

import jax
import jax.numpy as jnp
from jax import lax
from jax.experimental import pallas as pl
from jax.experimental.pallas import tpu as pltpu
from jax.experimental.pallas import tpu_sc as plsc

N = 10000
NPAD = 10240
E = 320000
D = 64
IN_D = 128
NW = 32
CH = 128
NBLK = E // CH
BCH = 78
G = 3
NGRP = BCH // G
RPT = 640

_f32 = jnp.float32


def _proj1_body(x_ref, w_ref, b_ref, p_ref, r_ref):
    xw = jnp.dot(x_ref[...], w_ref[...], preferred_element_type=_f32)
    p_ref[...] = xw[:, :D]
    r_ref[...] = xw[:, D:] + b_ref[...]


def _mid_body(s_ref, d_ref, r1_ref, w_ref, b_ref, p_ref, r_ref):
    s = s_ref[0, :N, :] + s_ref[1, :N, :]
    deg = (d_ref[0, :] + d_ref[1, :])[:N].reshape(N, 1)
    agg = s / jnp.maximum(deg, 1.0)
    h1 = jnp.maximum(agg + r1_ref[...], 0.0)
    hw = jnp.dot(h1, w_ref[...], preferred_element_type=_f32)
    p_ref[...] = hw[:, :D]
    r_ref[...] = hw[:, D:] + b_ref[...]


def _fin_body(s_ref, d_ref, r2_ref, o_ref):
    s = s_ref[0, :N, :] + s_ref[1, :N, :]
    deg = (d_ref[0, :] + d_ref[1, :])[:N].reshape(N, 1)
    agg = s / jnp.maximum(deg, 1.0)
    o_ref[...] = jnp.maximum(agg + r2_ref[...], 0.0)


_proj1 = pl.pallas_call(
    _proj1_body,
    out_shape=(jax.ShapeDtypeStruct((N, D), _f32),
               jax.ShapeDtypeStruct((N, D), _f32)),
)

_mid = pl.pallas_call(
    _mid_body,
    out_shape=(jax.ShapeDtypeStruct((N, D), _f32),
               jax.ShapeDtypeStruct((N, D), _f32)),
)

_fin = pl.pallas_call(
    _fin_body,
    out_shape=jax.ShapeDtypeStruct((N, D), _f32),
)


def _make_sc_pass(compute_deg):
    mesh = plsc.VectorSubcoreMesh(core_axis_name="c", subcore_axis_name="s")
    out_type = [jax.ShapeDtypeStruct((2, NPAD, D), _f32)]
    scratch = [
        pltpu.VMEM((BCH + 1, 2, CH), jnp.int32),
        pltpu.VMEM((2 * G, CH, D), _f32),
        pltpu.VMEM((32, D), _f32),
        pltpu.VMEM_SHARED((NPAD, D), _f32),
        pltpu.SemaphoreType.DMA,
        pltpu.SemaphoreType.DMA,
    ]
    if compute_deg:
        out_type.append(jax.ShapeDtypeStruct((2, NPAD), _f32))
        scratch += [
            pltpu.VMEM((CH,), _f32),
            pltpu.VMEM((RPT,), _f32),
            pltpu.VMEM_SHARED((NPAD,), _f32),
        ]

    def body(p_hbm, e_hbm, s_out, *rest):
        if compute_deg:
            (deg_out, idx_v, rows_v, zbuf, acc_sh, sem_a, sem_b,
             ones_v, dzbuf, deg_sh) = rest
        else:
            idx_v, rows_v, zbuf, acc_sh, sem_a, sem_b = rest
        cid = lax.axis_index("c")
        sid = lax.axis_index("s")
        wid = sid * 2 + cid
        base = sid * RPT
        cbase = wid * BCH + jnp.maximum(wid - 28, 0)

        idx_cp = pltpu.async_copy(e_hbm.at[pl.ds(cbase, BCH + 1)], idx_v,
                                  sem_a)

        def zrow(r, carry):
            for c in range(D // 16):
                zbuf[r, pl.ds(c * 16, 16)] = jnp.zeros((16,), _f32)
            return carry
        lax.fori_loop(0, 32, zrow, 0)
        for k in range(RPT // 32):
            pltpu.sync_copy(zbuf, acc_sh.at[pl.ds(base + k * 32, 32)])
        if compute_deg:
            def zdeg(i, carry):
                dzbuf[pl.ds(i * 16, 16)] = jnp.zeros((16,), _f32)
                return carry
            lax.fori_loop(0, RPT // 16, zdeg, 0)
            for i in range(CH // 16):
                ones_v[pl.ds(i * 16, 16)] = jnp.ones((16,), _f32)
            pltpu.sync_copy(dzbuf, deg_sh.at[pl.ds(base, RPT)])
        idx_cp.wait()
        plsc.subcore_barrier()

        def fire(g, boff, sem):
            for b in range(G):
                pltpu.async_copy(p_hbm.at[idx_v.at[g * G + b, 0]],
                                 rows_v.at[boff + b], sem)

        def drain_and_scatter(g, boff, sem):
            for b in range(G):
                pltpu.make_async_copy(p_hbm.at[idx_v.at[g * G + b, 0]],
                                      rows_v.at[boff + b], sem).wait()
            for b in range(G):
                c = g * G + b
                pltpu.sync_copy(rows_v.at[boff + b], acc_sh.at[idx_v.at[c, 1]],
                                add=True)
                if compute_deg:
                    pltpu.sync_copy(ones_v, deg_sh.at[idx_v.at[c, 1]],
                                    add=True)

        fire(0, 0, sem_a)

        def pair(t, carry):
            ga = 2 * t
            fire(ga + 1, G, sem_b)
            drain_and_scatter(ga, 0, sem_a)
            fire(ga + 2, 0, sem_a)
            drain_and_scatter(ga + 1, G, sem_b)
            return carry
        lax.fori_loop(0, (NGRP - 1) // 2, pair, 0)
        if NGRP % 2 == 1:
            drain_and_scatter(NGRP - 1, 0, sem_a)
        else:
            fire(NGRP - 1, G, sem_b)
            drain_and_scatter(NGRP - 2, 0, sem_a)
            drain_and_scatter(NGRP - 1, G, sem_b)

        @pl.when(wid >= 28)
        def _extra():
            pltpu.async_copy(p_hbm.at[idx_v.at[BCH, 0]], rows_v.at[0],
                             sem_a).wait()
            pltpu.sync_copy(rows_v.at[0], acc_sh.at[idx_v.at[BCH, 1]],
                            add=True)
            if compute_deg:
                pltpu.sync_copy(ones_v, deg_sh.at[idx_v.at[BCH, 1]], add=True)

        plsc.subcore_barrier()
        pltpu.sync_copy(acc_sh.at[pl.ds(base, RPT)],
                        s_out.at[cid, pl.ds(base, RPT)])
        if compute_deg:
            pltpu.sync_copy(deg_sh.at[pl.ds(base, RPT)],
                            deg_out.at[cid, pl.ds(base, RPT)])

    return pl.kernel(body, mesh=mesh, out_type=tuple(out_type),
                     scratch_types=scratch,
                     compiler_params=pltpu.CompilerParams(
                         use_tc_tiling_on_sc=False))


_sc_pass_deg = _make_sc_pass(True)
_sc_pass = _make_sc_pass(False)


def kernel(x, edge_index, W_l1, W_r1, b1, W_l2, W_r2, b2):
    ev = jnp.transpose(edge_index.astype(jnp.int32).reshape(2, NBLK, CH),
                       (1, 0, 2))
    wcat1 = jnp.concatenate([W_l1, W_r1], axis=1)
    wcat2 = jnp.concatenate([W_l2, W_r2], axis=1)

    p1, r1 = _proj1(x, wcat1, b1.reshape(1, D))
    s1, degp = _sc_pass_deg(p1, ev)
    p2, r2 = _mid(s1, degp, r1, wcat2, b2.reshape(1, D))
    s2, = _sc_pass(p2, ev)
    return _fin(s2, degp, r2)


# speedup vs baseline: 17.8467x; 1.0017x over previous
import functools

import jax
import jax.numpy as jnp
from jax import lax
from jax.experimental import pallas as pl
from jax.experimental.pallas import tpu as pltpu
from jax.experimental.pallas import tpu_sc as plsc

N = 10000
NPAD = 10240
E = 320000
D = 64
IN_D = 128
NW = 32
CH = 128
NBLK = E // CH
BCH = 78
G = 3
NGRP = BCH // G
RPT = 640

_f32 = jnp.float32


def _proj1_body(x_ref, w_ref, b_ref, p_ref, r_ref):
    xw = jnp.dot(x_ref[...], w_ref[...], preferred_element_type=_f32)
    p_ref[...] = xw[:, :D]
    r_ref[...] = xw[:, D:] + b_ref[...]


def _mid_body(s_ref, d_ref, r1_ref, w_ref, b_ref, p_ref, r_ref):
    s = s_ref[0, :N, :] + s_ref[1, :N, :]
    deg = (d_ref[0, :] + d_ref[1, :])[:N].reshape(N, 1)
    agg = s / jnp.maximum(deg, 1.0)
    h1 = jnp.maximum(agg + r1_ref[...], 0.0)
    hw = jnp.dot(h1, w_ref[...], preferred_element_type=_f32)
    p_ref[...] = hw[:, :D]
    r_ref[...] = hw[:, D:] + b_ref[...]


def _fin_body(s_ref, d_ref, r2_ref, o_ref):
    s = s_ref[0, :N, :] + s_ref[1, :N, :]
    deg = (d_ref[0, :] + d_ref[1, :])[:N].reshape(N, 1)
    agg = s / jnp.maximum(deg, 1.0)
    o_ref[...] = jnp.maximum(agg + r2_ref[...], 0.0)


_proj1 = pl.pallas_call(
    _proj1_body,
    out_shape=(jax.ShapeDtypeStruct((N, D), _f32),
               jax.ShapeDtypeStruct((N, D), _f32)),
)

_mid = pl.pallas_call(
    _mid_body,
    out_shape=(jax.ShapeDtypeStruct((N, D), _f32),
               jax.ShapeDtypeStruct((N, D), _f32)),
)

_fin = pl.pallas_call(
    _fin_body,
    out_shape=jax.ShapeDtypeStruct((N, D), _f32),
)


def _make_sc_pass(compute_deg):
    mesh = plsc.VectorSubcoreMesh(core_axis_name="c", subcore_axis_name="s")
    out_type = [jax.ShapeDtypeStruct((2, NPAD, D), _f32)]
    scratch = [
        pltpu.VMEM((BCH + 1, 2, CH), jnp.int32),
        pltpu.VMEM((2 * G, CH, D), _f32),
        pltpu.VMEM((32, D), _f32),
        pltpu.VMEM_SHARED((NPAD, D), _f32),
        pltpu.SemaphoreType.DMA,
        pltpu.SemaphoreType.DMA,
        pltpu.SemaphoreType.DMA,
    ]
    if compute_deg:
        out_type.append(jax.ShapeDtypeStruct((2, NPAD), _f32))
        scratch += [
            pltpu.VMEM((CH,), _f32),
            pltpu.VMEM((RPT,), _f32),
            pltpu.VMEM_SHARED((NPAD,), _f32),
        ]

    def body(p_hbm, e_hbm, s_out, *rest):
        if compute_deg:
            (deg_out, idx_v, rows_v, zbuf, acc_sh, sem_a, sem_b, sem_s,
             ones_v, dzbuf, deg_sh) = rest
        else:
            idx_v, rows_v, zbuf, acc_sh, sem_a, sem_b, sem_s = rest
        cid = lax.axis_index("c")
        sid = lax.axis_index("s")
        wid = sid * 2 + cid
        base = sid * RPT
        cbase = wid * BCH + jnp.maximum(wid - 28, 0)

        idx_cp = pltpu.async_copy(e_hbm.at[pl.ds(cbase, BCH + 1)], idx_v,
                                  sem_a)

        def zrow(r, carry):
            for c in range(D // 16):
                zbuf[r, pl.ds(c * 16, 16)] = jnp.zeros((16,), _f32)
            return carry
        lax.fori_loop(0, 32, zrow, 0)
        for k in range(RPT // 32):
            pltpu.sync_copy(zbuf, acc_sh.at[pl.ds(base + k * 32, 32)])
        if compute_deg:
            def zdeg(i, carry):
                dzbuf[pl.ds(i * 16, 16)] = jnp.zeros((16,), _f32)
                return carry
            lax.fori_loop(0, RPT // 16, zdeg, 0)
            for i in range(CH // 16):
                ones_v[pl.ds(i * 16, 16)] = jnp.ones((16,), _f32)
            pltpu.sync_copy(dzbuf, deg_sh.at[pl.ds(base, RPT)])
        idx_cp.wait()
        plsc.subcore_barrier()

        def fire(g, boff, sem):
            for b in range(G):
                pltpu.async_copy(p_hbm.at[idx_v.at[g * G + b, 0]],
                                 rows_v.at[boff + b], sem)

        def drain_and_scatter(g, boff, sem):
            for b in range(G):
                pltpu.make_async_copy(p_hbm.at[idx_v.at[g * G + b, 0]],
                                      rows_v.at[boff + b], sem).wait()
            for b in range(G):
                c = g * G + b
                pltpu.async_copy(rows_v.at[boff + b],
                                 acc_sh.at[idx_v.at[c, 1]], sem_s, add=True)
                if compute_deg:
                    pltpu.async_copy(ones_v, deg_sh.at[idx_v.at[c, 1]],
                                     sem_s, add=True)
            for b in range(G):
                c = g * G + b
                pltpu.make_async_copy(rows_v.at[boff + b],
                                      acc_sh.at[idx_v.at[c, 1]], sem_s).wait()
                if compute_deg:
                    pltpu.make_async_copy(ones_v, deg_sh.at[idx_v.at[c, 1]],
                                          sem_s).wait()

        fire(0, 0, sem_a)

        def pair(t, carry):
            ga = 2 * t
            fire(ga + 1, G, sem_b)
            drain_and_scatter(ga, 0, sem_a)
            fire(ga + 2, 0, sem_a)
            drain_and_scatter(ga + 1, G, sem_b)
            return carry
        lax.fori_loop(0, (NGRP - 1) // 2, pair, 0)
        if NGRP % 2 == 1:
            drain_and_scatter(NGRP - 1, 0, sem_a)
        else:
            fire(NGRP - 1, G, sem_b)
            drain_and_scatter(NGRP - 2, 0, sem_a)
            drain_and_scatter(NGRP - 1, G, sem_b)

        @pl.when(wid >= 28)
        def _extra():
            pltpu.async_copy(p_hbm.at[idx_v.at[BCH, 0]], rows_v.at[0],
                             sem_a).wait()
            pltpu.sync_copy(rows_v.at[0], acc_sh.at[idx_v.at[BCH, 1]],
                            add=True)
            if compute_deg:
                pltpu.sync_copy(ones_v, deg_sh.at[idx_v.at[BCH, 1]], add=True)

        plsc.subcore_barrier()
        pltpu.sync_copy(acc_sh.at[pl.ds(base, RPT)],
                        s_out.at[cid, pl.ds(base, RPT)])
        if compute_deg:
            pltpu.sync_copy(deg_sh.at[pl.ds(base, RPT)],
                            deg_out.at[cid, pl.ds(base, RPT)])

    return pl.kernel(body, mesh=mesh, out_type=tuple(out_type),
                     scratch_types=scratch,
                     compiler_params=pltpu.CompilerParams(
                         use_tc_tiling_on_sc=False))


_sc_pass_deg = _make_sc_pass(True)
_sc_pass = _make_sc_pass(False)


def kernel(x, edge_index, W_l1, W_r1, b1, W_l2, W_r2, b2):
    ev = jnp.transpose(edge_index.astype(jnp.int32).reshape(2, NBLK, CH),
                       (1, 0, 2))
    wcat1 = jnp.concatenate([W_l1, W_r1], axis=1)
    wcat2 = jnp.concatenate([W_l2, W_r2], axis=1)

    p1, r1 = _proj1(x, wcat1, b1.reshape(1, D))
    s1, degp = _sc_pass_deg(p1, ev)
    p2, r2 = _mid(s1, degp, r1, wcat2, b2.reshape(1, D))
    s2, = _sc_pass(p2, ev)
    return _fin(s2, degp, r2)


# speedup vs baseline: 20.8224x; 1.1667x over previous
import functools

import jax
import jax.numpy as jnp
from jax import lax
from jax.experimental import pallas as pl
from jax.experimental.pallas import tpu as pltpu
from jax.experimental.pallas import tpu_sc as plsc

N = 10000
NPAD = 10240
E = 320000
D = 64
IN_D = 128
NW = 32
CH = 128
NBLK = E // CH
BCH = 78
G = 3
NGRP = BCH // G
RPT = 640

_f32 = jnp.float32


H = NPAD // 2
NR = N - H
PADR = H - NR


def _proj1_body(x_ref, w_ref, b_ref, p_ref, r_ref):
    xw = jnp.dot(x_ref[...], w_ref[...], preferred_element_type=_f32)
    p_ref[:, :D] = xw[:H, :D]
    p_ref[:NR, D:] = xw[H:, :D]
    p_ref[NR:, D:] = jnp.zeros((PADR, D), _f32)
    r_ref[:, :D] = xw[:H, D:] + b_ref[:, :D]
    r_ref[:NR, D:] = xw[H:, D:] + b_ref[:, D:]
    r_ref[NR:, D:] = jnp.zeros((PADR, D), _f32)


def _mid_body(s_ref, d_ref, r1_ref, w_ref, b_ref, sel_ref, p_ref, r_ref,
              rp_ref):
    ss = s_ref[0] + s_ref[1]
    dsum = d_ref[0] + d_ref[1]
    degq = jnp.dot(dsum, sel_ref[...], preferred_element_type=_f32)
    rp = 1.0 / jnp.maximum(degq, 1.0)
    rp_ref[...] = rp
    h1 = jnp.maximum(ss * rp + r1_ref[...], 0.0)
    hw = jnp.dot(h1, w_ref[...], preferred_element_type=_f32)
    p_ref[...] = hw[:, :2 * D]
    r_ref[...] = hw[:, 2 * D:] + b_ref[...]


def _fin_body(s_ref, rp_ref, r2_ref, o_ref):
    ss = s_ref[0] + s_ref[1]
    res = jnp.maximum(ss * rp_ref[...] + r2_ref[...], 0.0)
    o_ref[:H, :] = res[:, :D]
    o_ref[H:, :] = res[:NR, D:]


_proj1 = pl.pallas_call(
    _proj1_body,
    out_shape=(jax.ShapeDtypeStruct((H, 2 * D), _f32),
               jax.ShapeDtypeStruct((H, 2 * D), _f32)),
)

_mid = pl.pallas_call(
    _mid_body,
    out_shape=(jax.ShapeDtypeStruct((H, 2 * D), _f32),
               jax.ShapeDtypeStruct((H, 2 * D), _f32),
               jax.ShapeDtypeStruct((H, 2 * D), _f32)),
)

_fin = pl.pallas_call(
    _fin_body,
    out_shape=jax.ShapeDtypeStruct((N, D), _f32),
)


def _make_sc_pass(compute_deg):
    mesh = plsc.VectorSubcoreMesh(core_axis_name="c", subcore_axis_name="s")
    out_type = [jax.ShapeDtypeStruct((2, NPAD, D), _f32)]
    scratch = [
        pltpu.VMEM((BCH + 1, 2, CH), jnp.int32),
        pltpu.VMEM((2 * G, CH, D), _f32),
        pltpu.VMEM((32, D), _f32),
        pltpu.VMEM_SHARED((NPAD, D), _f32),
        pltpu.SemaphoreType.DMA,
        pltpu.SemaphoreType.DMA,
        pltpu.SemaphoreType.DMA,
    ]
    if compute_deg:
        out_type.append(jax.ShapeDtypeStruct((2, NPAD), _f32))
        scratch += [
            pltpu.VMEM((CH,), _f32),
            pltpu.VMEM((RPT,), _f32),
            pltpu.VMEM_SHARED((NPAD,), _f32),
        ]

    def body(p_hbm, e_hbm, s_out, *rest):
        if compute_deg:
            (deg_out, idx_v, rows_v, zbuf, acc_sh, sem_a, sem_b, sem_s,
             ones_v, dzbuf, deg_sh) = rest
        else:
            idx_v, rows_v, zbuf, acc_sh, sem_a, sem_b, sem_s = rest
        cid = lax.axis_index("c")
        sid = lax.axis_index("s")
        wid = sid * 2 + cid
        base = sid * RPT
        cbase = wid * BCH + jnp.maximum(wid - 28, 0)

        idx_cp = pltpu.async_copy(e_hbm.at[pl.ds(cbase, BCH + 1)], idx_v,
                                  sem_a)

        def zrow(r, carry):
            for c in range(D // 16):
                zbuf[r, pl.ds(c * 16, 16)] = jnp.zeros((16,), _f32)
            return carry
        lax.fori_loop(0, 32, zrow, 0)
        for k in range(RPT // 32):
            pltpu.sync_copy(zbuf, acc_sh.at[pl.ds(base + k * 32, 32)])
        if compute_deg:
            def zdeg(i, carry):
                dzbuf[pl.ds(i * 16, 16)] = jnp.zeros((16,), _f32)
                return carry
            lax.fori_loop(0, RPT // 16, zdeg, 0)
            for i in range(CH // 16):
                ones_v[pl.ds(i * 16, 16)] = jnp.ones((16,), _f32)
            pltpu.sync_copy(dzbuf, deg_sh.at[pl.ds(base, RPT)])
        idx_cp.wait()
        plsc.subcore_barrier()

        def remap_row(c):
            for h2 in range(2):
                for v in range(CH // 16):
                    sl = pl.ds(v * 16, 16)
                    t = idx_v[c, h2, sl]
                    idx_v[c, h2, sl] = jnp.where(t >= H, t * 2 - (2 * H - 1),
                                                 t * 2)

        def remap(g):
            for b in range(G):
                remap_row(g * G + b)

        def fire(g, boff, sem):
            for b in range(G):
                pltpu.async_copy(p_hbm.at[idx_v.at[g * G + b, 0]],
                                 rows_v.at[boff + b], sem)

        def drain_and_scatter(g, boff, sem):
            for b in range(G):
                pltpu.make_async_copy(p_hbm.at[idx_v.at[g * G + b, 0]],
                                      rows_v.at[boff + b], sem).wait()
            for b in range(G):
                c = g * G + b
                pltpu.async_copy(rows_v.at[boff + b],
                                 acc_sh.at[idx_v.at[c, 1]], sem_s, add=True)
                if compute_deg:
                    pltpu.async_copy(ones_v, deg_sh.at[idx_v.at[c, 1]],
                                     sem_s, add=True)
            for b in range(G):
                c = g * G + b
                pltpu.make_async_copy(rows_v.at[boff + b],
                                      acc_sh.at[idx_v.at[c, 1]], sem_s).wait()
                if compute_deg:
                    pltpu.make_async_copy(ones_v, deg_sh.at[idx_v.at[c, 1]],
                                          sem_s).wait()

        remap(0)
        fire(0, 0, sem_a)
        remap(1)

        def pair(t, carry):
            ga = 2 * t
            fire(ga + 1, G, sem_b)
            remap(ga + 2)
            drain_and_scatter(ga, 0, sem_a)
            fire(ga + 2, 0, sem_a)
            remap(ga + 3)
            drain_and_scatter(ga + 1, G, sem_b)
            return carry
        lax.fori_loop(0, (NGRP - 1) // 2, pair, 0)
        if NGRP % 2 == 1:
            drain_and_scatter(NGRP - 1, 0, sem_a)
        else:
            fire(NGRP - 1, G, sem_b)
            drain_and_scatter(NGRP - 2, 0, sem_a)
            drain_and_scatter(NGRP - 1, G, sem_b)

        @pl.when(wid >= 28)
        def _extra():
            remap_row(BCH)
            pltpu.async_copy(p_hbm.at[idx_v.at[BCH, 0]], rows_v.at[0],
                             sem_a).wait()
            pltpu.sync_copy(rows_v.at[0], acc_sh.at[idx_v.at[BCH, 1]],
                            add=True)
            if compute_deg:
                pltpu.sync_copy(ones_v, deg_sh.at[idx_v.at[BCH, 1]], add=True)

        plsc.subcore_barrier()
        pltpu.sync_copy(acc_sh.at[pl.ds(base, RPT)],
                        s_out.at[cid, pl.ds(base, RPT)])
        if compute_deg:
            pltpu.sync_copy(deg_sh.at[pl.ds(base, RPT)],
                            deg_out.at[cid, pl.ds(base, RPT)])

    return pl.kernel(body, mesh=mesh, out_type=tuple(out_type),
                     scratch_types=scratch,
                     compiler_params=pltpu.CompilerParams(
                         use_tc_tiling_on_sc=False))


_sc_pass_deg = _make_sc_pass(True)
_sc_pass = _make_sc_pass(False)


def kernel(x, edge_index, W_l1, W_r1, b1, W_l2, W_r2, b2):
    ev = jnp.transpose(edge_index.astype(jnp.int32).reshape(2, NBLK, CH),
                       (1, 0, 2))
    wcat1 = jnp.concatenate([W_l1, W_r1], axis=1)
    eye2 = jnp.eye(2, dtype=_f32)
    wbig2 = jnp.concatenate([jnp.kron(eye2, W_l2), jnp.kron(eye2, W_r2)],
                            axis=1)
    bsel = jnp.kron(eye2, jnp.ones((1, D), _f32))
    b1_blk = jnp.tile(b1.reshape(1, D), (1, 2))
    b2_blk = jnp.tile(b2.reshape(1, D), (1, 2))

    p1, r1 = _proj1(x, wcat1, b1_blk)
    s1, degp = _sc_pass_deg(p1.reshape(NPAD, D), ev)
    p2, r2, rp = _mid(s1.reshape(2, H, 2 * D), degp.reshape(2, H, 2), r1,
                      wbig2, b2_blk, bsel)
    s2, = _sc_pass(p2.reshape(NPAD, D), ev)
    return _fin(s2.reshape(2, H, 2 * D), rp, r2)


# speedup vs baseline: 21.7717x; 1.0456x over previous
import functools

import jax
import jax.numpy as jnp
from jax import lax
from jax.experimental import pallas as pl
from jax.experimental.pallas import tpu as pltpu
from jax.experimental.pallas import tpu_sc as plsc

N = 10000
NPAD = 10240
E = 320000
D = 64
IN_D = 128
NW = 32
CH = 128
NBLK = E // CH
BCH = 78
G = 3
NGRP = BCH // G
RPT = 640

_f32 = jnp.float32


H = NPAD // 2
NR = N - H
PADR = H - NR


def _proj1_body(x_ref, w_ref, b_ref, p_ref, r_ref):
    xw = jnp.dot(x_ref[...], w_ref[...], preferred_element_type=_f32)
    p_ref[:, :D] = xw[:H, :D]
    p_ref[:NR, D:] = xw[H:, :D]
    p_ref[NR:, D:] = jnp.zeros((PADR, D), _f32)
    r_ref[:, :D] = xw[:H, D:] + b_ref[:, :D]
    r_ref[:NR, D:] = xw[H:, D:] + b_ref[:, D:]
    r_ref[NR:, D:] = jnp.zeros((PADR, D), _f32)


def _mid_body(s_ref, d_ref, r1_ref, w_ref, b_ref, p_ref, r_ref,
              rp_ref):
    ss = s_ref[0] + s_ref[1]
    dsum = d_ref[0, :] + d_ref[1, :]
    rp_l = 1.0 / jnp.maximum(dsum[:H].reshape(H, 1), 1.0)
    rp_r = 1.0 / jnp.maximum(dsum[H:].reshape(H, 1), 1.0)
    rp = jnp.concatenate([jnp.broadcast_to(rp_l, (H, D)),
                          jnp.broadcast_to(rp_r, (H, D))], axis=1)
    rp_ref[...] = rp
    h1 = jnp.maximum(ss * rp + r1_ref[...], 0.0)
    hw = jnp.dot(h1, w_ref[...], preferred_element_type=_f32)
    p_ref[...] = hw[:, :2 * D]
    r_ref[...] = hw[:, 2 * D:] + b_ref[...]


def _fin_body(s_ref, rp_ref, r2_ref, o_ref):
    ss = s_ref[0] + s_ref[1]
    res = jnp.maximum(ss * rp_ref[...] + r2_ref[...], 0.0)
    o_ref[:H, :] = res[:, :D]
    o_ref[H:, :] = res[:NR, D:]


_proj1 = pl.pallas_call(
    _proj1_body,
    out_shape=(jax.ShapeDtypeStruct((H, 2 * D), _f32),
               jax.ShapeDtypeStruct((H, 2 * D), _f32)),
)

_mid = pl.pallas_call(
    _mid_body,
    out_shape=(jax.ShapeDtypeStruct((H, 2 * D), _f32),
               jax.ShapeDtypeStruct((H, 2 * D), _f32),
               jax.ShapeDtypeStruct((H, 2 * D), _f32)),
)

_fin = pl.pallas_call(
    _fin_body,
    out_shape=jax.ShapeDtypeStruct((N, D), _f32),
)


def _make_sc_pass(compute_deg):
    mesh = plsc.VectorSubcoreMesh(core_axis_name="c", subcore_axis_name="s")
    out_type = [jax.ShapeDtypeStruct((2, NPAD, D), _f32)]
    scratch = [
        pltpu.VMEM((BCH + 1, 2, CH), jnp.int32),
        pltpu.VMEM((BCH + 1, CH), jnp.int32),
        pltpu.VMEM((2 * G, CH, D), _f32),
        pltpu.VMEM((32, D), _f32),
        pltpu.VMEM_SHARED((NPAD, D), _f32),
        pltpu.SemaphoreType.DMA,
        pltpu.SemaphoreType.DMA,
        pltpu.SemaphoreType.DMA,
    ]
    if compute_deg:
        out_type.append(jax.ShapeDtypeStruct((2, NPAD), _f32))
        scratch += [
            pltpu.VMEM((CH,), _f32),
            pltpu.VMEM((RPT,), _f32),
            pltpu.VMEM_SHARED((NPAD,), _f32),
        ]

    def body(p_hbm, e_hbm, s_out, *rest):
        if compute_deg:
            (deg_out, idx_v, dstr_v, rows_v, zbuf, acc_sh, sem_a, sem_b,
             sem_s, ones_v, dzbuf, deg_sh) = rest
        else:
            idx_v, dstr_v, rows_v, zbuf, acc_sh, sem_a, sem_b, sem_s = rest
        cid = lax.axis_index("c")
        sid = lax.axis_index("s")
        wid = sid * 2 + cid
        base = sid * RPT
        cbase = wid * BCH + jnp.maximum(wid - 28, 0)

        idx_cp = pltpu.async_copy(e_hbm.at[pl.ds(cbase, BCH + 1)], idx_v,
                                  sem_a)

        def zrow(r, carry):
            for c in range(D // 16):
                zbuf[r, pl.ds(c * 16, 16)] = jnp.zeros((16,), _f32)
            return carry
        lax.fori_loop(0, 32, zrow, 0)
        for k in range(RPT // 32):
            pltpu.sync_copy(zbuf, acc_sh.at[pl.ds(base + k * 32, 32)])
        if compute_deg:
            def zdeg(i, carry):
                dzbuf[pl.ds(i * 16, 16)] = jnp.zeros((16,), _f32)
                return carry
            lax.fori_loop(0, RPT // 16, zdeg, 0)
            for i in range(CH // 16):
                ones_v[pl.ds(i * 16, 16)] = jnp.ones((16,), _f32)
            pltpu.sync_copy(dzbuf, deg_sh.at[pl.ds(base, RPT)])
        idx_cp.wait()
        plsc.subcore_barrier()

        def remap_row(c):
            for v in range(CH // 16):
                sl = pl.ds(v * 16, 16)
                t = idx_v[c, 0, sl]
                idx_v[c, 0, sl] = jnp.where(t >= H, t * 2 - (2 * H - 1),
                                            t * 2)
                u = idx_v[c, 1, sl]
                dstr_v[c, sl] = jnp.where(u >= H, u * 2 - (2 * H - 1), u * 2)

        def remap(g):
            for b in range(G):
                remap_row(g * G + b)

        def fire(g, boff, sem):
            for b in range(G):
                pltpu.async_copy(p_hbm.at[idx_v.at[g * G + b, 0]],
                                 rows_v.at[boff + b], sem)

        def drain_and_scatter(g, boff, sem):
            for b in range(G):
                pltpu.make_async_copy(p_hbm.at[idx_v.at[g * G + b, 0]],
                                      rows_v.at[boff + b], sem).wait()
            for b in range(G):
                c = g * G + b
                pltpu.async_copy(rows_v.at[boff + b],
                                 acc_sh.at[dstr_v.at[c]], sem_s, add=True)
                if compute_deg:
                    pltpu.async_copy(ones_v, deg_sh.at[idx_v.at[c, 1]],
                                     sem_s, add=True)
            for b in range(G):
                c = g * G + b
                pltpu.make_async_copy(rows_v.at[boff + b],
                                      acc_sh.at[dstr_v.at[c]], sem_s).wait()
                if compute_deg:
                    pltpu.make_async_copy(ones_v, deg_sh.at[idx_v.at[c, 1]],
                                          sem_s).wait()

        remap(0)
        fire(0, 0, sem_a)
        remap(1)

        def pair(t, carry):
            ga = 2 * t
            fire(ga + 1, G, sem_b)
            remap(ga + 2)
            drain_and_scatter(ga, 0, sem_a)
            fire(ga + 2, 0, sem_a)
            remap(ga + 3)
            drain_and_scatter(ga + 1, G, sem_b)
            return carry
        lax.fori_loop(0, (NGRP - 1) // 2, pair, 0)
        if NGRP % 2 == 1:
            drain_and_scatter(NGRP - 1, 0, sem_a)
        else:
            fire(NGRP - 1, G, sem_b)
            drain_and_scatter(NGRP - 2, 0, sem_a)
            drain_and_scatter(NGRP - 1, G, sem_b)

        @pl.when(wid >= 28)
        def _extra():
            remap_row(BCH)
            pltpu.async_copy(p_hbm.at[idx_v.at[BCH, 0]], rows_v.at[0],
                             sem_a).wait()
            pltpu.sync_copy(rows_v.at[0], acc_sh.at[dstr_v.at[BCH]],
                            add=True)
            if compute_deg:
                pltpu.sync_copy(ones_v, deg_sh.at[idx_v.at[BCH, 1]], add=True)

        plsc.subcore_barrier()
        pltpu.sync_copy(acc_sh.at[pl.ds(base, RPT)],
                        s_out.at[cid, pl.ds(base, RPT)])
        if compute_deg:
            pltpu.sync_copy(deg_sh.at[pl.ds(base, RPT)],
                            deg_out.at[cid, pl.ds(base, RPT)])

    return pl.kernel(body, mesh=mesh, out_type=tuple(out_type),
                     scratch_types=scratch,
                     compiler_params=pltpu.CompilerParams(
                         use_tc_tiling_on_sc=False))


_sc_pass_deg = _make_sc_pass(True)
_sc_pass = _make_sc_pass(False)


def kernel(x, edge_index, W_l1, W_r1, b1, W_l2, W_r2, b2):
    ev = jnp.transpose(edge_index.astype(jnp.int32).reshape(2, NBLK, CH),
                       (1, 0, 2))
    wcat1 = jnp.concatenate([W_l1, W_r1], axis=1)
    eye2 = jnp.eye(2, dtype=_f32)
    wbig2 = jnp.concatenate([jnp.kron(eye2, W_l2), jnp.kron(eye2, W_r2)],
                            axis=1)
    b1_blk = jnp.tile(b1.reshape(1, D), (1, 2))
    b2_blk = jnp.tile(b2.reshape(1, D), (1, 2))

    p1, r1 = _proj1(x, wcat1, b1_blk)
    s1, degp = _sc_pass_deg(p1.reshape(NPAD, D), ev)
    p2, r2, rp = _mid(s1.reshape(2, H, 2 * D), degp, r1, wbig2, b2_blk)
    s2, = _sc_pass(p2.reshape(NPAD, D), ev)
    return _fin(s2.reshape(2, H, 2 * D), rp, r2)
